# Initial kernel scaffold; baseline (speedup 1.0000x reference)
import functools

import jax
import jax.numpy as jnp
from jax import lax
from jax.experimental import pallas as pl
from jax.experimental.pallas import tpu as pltpu
from jax.experimental.pallas import tpu_sc as plsc

VOCAB = 100000
EMBED = 128
HIDDEN = 2048
OUT = 1024
B = 16384
L = 50

NC = 2
NS = 16
NW = NC * NS
BPW = B // NW
BLK = 64
NBLK = BPW // BLK
NV = EMBED // 16


def _embed_pool_body(x_hbm, table_hbm, out_hbm, idx_v, rows_v, acc_v,
                     sem_x, sem_g, sem_o):
    wid = lax.axis_index("s") * NC + lax.axis_index("c")
    base = wid * BPW

    pltpu.make_async_copy(
        x_hbm.at[pl.ds(base, BLK)], idx_v.at[0], sem_x).start()

    for blk in range(NBLK):
        xbuf = blk % 2
        obuf = blk % 2
        bbase = base + blk * BLK

        pltpu.make_async_copy(
            x_hbm.at[pl.ds(bbase, BLK)], idx_v.at[xbuf], sem_x).wait()
        if blk + 1 < NBLK:
            pltpu.make_async_copy(
                x_hbm.at[pl.ds(bbase + BLK, BLK)], idx_v.at[1 - xbuf],
                sem_x).start()
        if blk >= 2:
            pltpu.make_async_copy(
                acc_v.at[obuf], out_hbm.at[pl.ds(base, BLK)], sem_o).wait()

        for j in range(2):
            pltpu.make_async_copy(
                table_hbm.at[idx_v.at[xbuf, j]], rows_v.at[j], sem_g).start()

        def body(i, carry, xbuf=xbuf, obuf=obuf):
            for j in range(2):
                bb = i * 2 + j
                pltpu.make_async_copy(
                    table_hbm.at[idx_v.at[xbuf, 0]], rows_v.at[j],
                    sem_g).wait()
                acc = [rows_v[j, 0, pl.ds(16 * v, 16)] for v in range(NV)]
                for el in range(1, L):
                    for v in range(NV):
                        acc[v] = acc[v] + rows_v[j, el, pl.ds(16 * v, 16)]
                for v in range(NV):
                    acc_v[obuf, bb, pl.ds(16 * v, 16)] = acc[v]

                @pl.when(bb + 2 < BLK)
                def _():
                    pltpu.make_async_copy(
                        table_hbm.at[idx_v.at[xbuf, bb + 2]], rows_v.at[j],
                        sem_g).start()
            return carry

        lax.fori_loop(0, BLK // 2, body, 0)

        pltpu.make_async_copy(
            acc_v.at[obuf], out_hbm.at[pl.ds(bbase, BLK)], sem_o).start()

    for _ in range(2):
        pltpu.make_async_copy(
            acc_v.at[0], out_hbm.at[pl.ds(base, BLK)], sem_o).wait()


_embed_pool = functools.partial(
    pl.kernel,
    mesh=plsc.VectorSubcoreMesh(core_axis_name="c", subcore_axis_name="s"),
    out_type=jax.ShapeDtypeStruct((B, EMBED), jnp.float32),
    scratch_types=[
        pltpu.VMEM((2, BLK, L), jnp.int32),
        pltpu.VMEM((2, L, EMBED), jnp.float32),
        pltpu.VMEM((2, BLK, EMBED), jnp.float32),
        pltpu.SemaphoreType.DMA,
        pltpu.SemaphoreType.DMA,
        pltpu.SemaphoreType.DMA,
    ],
)(_embed_pool_body)


BM = 512


def _mlp_kernel(x_ref, w1_ref, b1_ref, w2_ref, b2_ref, o_ref):
    x = x_ref[...] * (1.0 / L)
    h = jnp.dot(x, w1_ref[...], preferred_element_type=jnp.float32)
    h = jnp.maximum(h + b1_ref[...], 0.0)
    o = jnp.dot(h, w2_ref[...], preferred_element_type=jnp.float32)
    o_ref[...] = o + b2_ref[...]


def _mlp(pooled_sum, W1, b1, W2, b2):
    return pl.pallas_call(
        _mlp_kernel,
        grid=(B // BM,),
        in_specs=[
            pl.BlockSpec((BM, EMBED), lambda i: (i, 0)),
            pl.BlockSpec((EMBED, HIDDEN), lambda i: (0, 0)),
            pl.BlockSpec((1, HIDDEN), lambda i: (0, 0)),
            pl.BlockSpec((HIDDEN, OUT), lambda i: (0, 0)),
            pl.BlockSpec((1, OUT), lambda i: (0, 0)),
        ],
        out_specs=pl.BlockSpec((BM, OUT), lambda i: (i, 0)),
        out_shape=jax.ShapeDtypeStruct((B, OUT), jnp.float32),
    )(pooled_sum, W1, b1.reshape(1, HIDDEN), W2, b2.reshape(1, OUT))


def kernel(x, table, W1, b1, W2, b2):
    pooled_sum = _embed_pool(x, table)
    return _mlp(pooled_sum, W1, b1, W2, b2)


# speedup vs baseline: 8.5934x; 8.5934x over previous
import functools

import jax
import jax.numpy as jnp
from jax import lax
from jax.experimental import pallas as pl
from jax.experimental.pallas import tpu as pltpu
from jax.experimental.pallas import tpu_sc as plsc

VOCAB = 100000
EMBED = 128
HIDDEN = 2048
OUT = 1024
B = 16384
L = 50

NC = 2
NS = 16
NW = NC * NS
BPW = B // NW
NV = EMBED // 16
UNR = 10


BLK = 64
NBLK2 = BPW // (2 * BLK)


def _embed_pool_body(x_hbm, table_hbm, out_hbm, idx_v, rows_v, acc_v,
                     sem_g, sem_o):
    wid = lax.axis_index("s") * NC + lax.axis_index("c")
    base = wid * BPW

    pltpu.sync_copy(x_hbm.at[pl.ds(base, BPW)], idx_v)

    for j in range(2):
        pltpu.make_async_copy(
            table_hbm.at[idx_v.at[j]], rows_v.at[j], sem_g).start()

    def blkbody(ib, carry):
        for k in range(2):
            blkbase = ib * (2 * BLK) + k * BLK

            @pl.when(ib > 0)
            def _(k=k):
                pltpu.make_async_copy(
                    acc_v.at[k], out_hbm.at[pl.ds(base, BLK)], sem_o).wait()

            def ibody(i2, c, k=k, blkbase=blkbase):
                for j in range(2):
                    off = i2 * 2 + j
                    bb = blkbase + off
                    pltpu.make_async_copy(
                        table_hbm.at[idx_v.at[0]], rows_v.at[j],
                        sem_g).wait()

                    def lbody(e, acc, j=j):
                        acc = list(acc)
                        for u in range(UNR):
                            for v in range(NV):
                                acc[v] = acc[v] + rows_v[j, e * UNR + u,
                                                         pl.ds(16 * v, 16)]
                        return acc

                    acc = lax.fori_loop(
                        0, L // UNR, lbody,
                        [jnp.zeros((16,), jnp.float32)] * NV)
                    for v in range(NV):
                        acc_v[k, off, pl.ds(16 * v, 16)] = acc[v]

                    @pl.when(bb + 2 < BPW)
                    def _(j=j, bb=bb):
                        pltpu.make_async_copy(
                            table_hbm.at[idx_v.at[bb + 2]], rows_v.at[j],
                            sem_g).start()
                return c

            lax.fori_loop(0, BLK // 2, ibody, 0)

            pltpu.make_async_copy(
                acc_v.at[k], out_hbm.at[pl.ds(base + blkbase, BLK)],
                sem_o).start()
        return carry

    lax.fori_loop(0, NBLK2, blkbody, 0)

    for k in range(2):
        pltpu.make_async_copy(
            acc_v.at[k], out_hbm.at[pl.ds(base, BLK)], sem_o).wait()


_embed_pool = functools.partial(
    pl.kernel,
    mesh=plsc.VectorSubcoreMesh(core_axis_name="c", subcore_axis_name="s"),
    out_type=jax.ShapeDtypeStruct((B, EMBED), jnp.float32),
    scratch_types=[
        pltpu.VMEM((BPW, L), jnp.int32),
        pltpu.VMEM((2, L, EMBED), jnp.float32),
        pltpu.VMEM((2, BLK, EMBED), jnp.float32),
        pltpu.SemaphoreType.DMA,
        pltpu.SemaphoreType.DMA,
    ],
)(_embed_pool_body)


BM = 512


def _mlp_kernel(x_ref, w1_ref, b1_ref, w2_ref, b2_ref, o_ref):
    x = x_ref[...] * (1.0 / L)
    h = jnp.dot(x, w1_ref[...], preferred_element_type=jnp.float32)
    h = jnp.maximum(h + b1_ref[...], 0.0)
    o = jnp.dot(h, w2_ref[...], preferred_element_type=jnp.float32)
    o_ref[...] = o + b2_ref[...]


def _mlp(pooled_sum, W1, b1, W2, b2):
    return pl.pallas_call(
        _mlp_kernel,
        grid=(B // BM,),
        in_specs=[
            pl.BlockSpec((BM, EMBED), lambda i: (i, 0)),
            pl.BlockSpec((EMBED, HIDDEN), lambda i: (0, 0)),
            pl.BlockSpec((1, HIDDEN), lambda i: (0, 0)),
            pl.BlockSpec((HIDDEN, OUT), lambda i: (0, 0)),
            pl.BlockSpec((1, OUT), lambda i: (0, 0)),
        ],
        out_specs=pl.BlockSpec((BM, OUT), lambda i: (i, 0)),
        out_shape=jax.ShapeDtypeStruct((B, OUT), jnp.float32),
    )(pooled_sum, W1, b1.reshape(1, HIDDEN), W2, b2.reshape(1, OUT))


def kernel(x, table, W1, b1, W2, b2):
    pooled_sum = _embed_pool(x, table)
    return _mlp(pooled_sum, W1, b1, W2, b2)


# speedup vs baseline: 12.2324x; 1.4235x over previous
import functools

import jax
import jax.numpy as jnp
from jax import lax
from jax.experimental import pallas as pl
from jax.experimental.pallas import tpu as pltpu
from jax.experimental.pallas import tpu_sc as plsc

VOCAB = 100000
EMBED = 128
HIDDEN = 2048
OUT = 1024
B = 16384
L = 50

NC = 2
NS = 16
NW = NC * NS
BPW = B // NW
NV = EMBED // 16
UNR = 10
NBUF = 4


BLK = 64
NBLK2 = BPW // (2 * BLK)


def _embed_pool_body(x_hbm, table_hbm, out_hbm, idx_v, rows_v, acc_v,
                     sem_g, sem_o):
    wid = lax.axis_index("s") * NC + lax.axis_index("c")
    base = wid * BPW

    pltpu.sync_copy(x_hbm.at[pl.ds(base, BPW)], idx_v)

    for j in range(NBUF):
        pltpu.make_async_copy(
            table_hbm.at[idx_v.at[j]], rows_v.at[j], sem_g).start()

    def blkbody(ib, carry):
        for k in range(2):
            blkbase = ib * (2 * BLK) + k * BLK

            @pl.when(ib > 0)
            def _(k=k):
                pltpu.make_async_copy(
                    acc_v.at[k], out_hbm.at[pl.ds(base, BLK)], sem_o).wait()

            def ibody(i2, c, k=k, blkbase=blkbase):
                for j in range(NBUF):
                    off = i2 * NBUF + j
                    bb = blkbase + off
                    pltpu.make_async_copy(
                        table_hbm.at[idx_v.at[0]], rows_v.at[j],
                        sem_g).wait()

                    def lbody(e, acc, j=j):
                        acc = list(acc)
                        for u in range(UNR):
                            for v in range(NV):
                                acc[v] = acc[v] + rows_v[j, e * UNR + u,
                                                         pl.ds(16 * v, 16)]
                        return acc

                    acc = lax.fori_loop(
                        0, L // UNR, lbody,
                        [jnp.zeros((16,), jnp.float32)] * NV)
                    for v in range(NV):
                        acc_v[k, off, pl.ds(16 * v, 16)] = acc[v]

                    @pl.when(bb + NBUF < BPW)
                    def _(j=j, bb=bb):
                        pltpu.make_async_copy(
                            table_hbm.at[idx_v.at[bb + NBUF]], rows_v.at[j],
                            sem_g).start()
                return c

            lax.fori_loop(0, BLK // NBUF, ibody, 0)

            pltpu.make_async_copy(
                acc_v.at[k], out_hbm.at[pl.ds(base + blkbase, BLK)],
                sem_o).start()
        return carry

    lax.fori_loop(0, NBLK2, blkbody, 0)

    for k in range(2):
        pltpu.make_async_copy(
            acc_v.at[k], out_hbm.at[pl.ds(base, BLK)], sem_o).wait()


_embed_pool = functools.partial(
    pl.kernel,
    mesh=plsc.VectorSubcoreMesh(core_axis_name="c", subcore_axis_name="s"),
    out_type=jax.ShapeDtypeStruct((B, EMBED), jnp.float32),
    scratch_types=[
        pltpu.VMEM((BPW, L), jnp.int32),
        pltpu.VMEM((NBUF, L, EMBED), jnp.float32),
        pltpu.VMEM((2, BLK, EMBED), jnp.float32),
        pltpu.SemaphoreType.DMA,
        pltpu.SemaphoreType.DMA,
    ],
)(_embed_pool_body)


BM = 512


def _mlp_kernel(x_ref, w1_ref, b1_ref, w2_ref, b2_ref, o_ref):
    x = x_ref[...] * (1.0 / L)
    h = jnp.dot(x, w1_ref[...], preferred_element_type=jnp.float32)
    h = jnp.maximum(h + b1_ref[...], 0.0)
    o = jnp.dot(h, w2_ref[...], preferred_element_type=jnp.float32)
    o_ref[...] = o + b2_ref[...]


def _mlp(pooled_sum, W1, b1, W2, b2):
    return pl.pallas_call(
        _mlp_kernel,
        grid=(B // BM,),
        in_specs=[
            pl.BlockSpec((BM, EMBED), lambda i: (i, 0)),
            pl.BlockSpec((EMBED, HIDDEN), lambda i: (0, 0)),
            pl.BlockSpec((1, HIDDEN), lambda i: (0, 0)),
            pl.BlockSpec((HIDDEN, OUT), lambda i: (0, 0)),
            pl.BlockSpec((1, OUT), lambda i: (0, 0)),
        ],
        out_specs=pl.BlockSpec((BM, OUT), lambda i: (i, 0)),
        out_shape=jax.ShapeDtypeStruct((B, OUT), jnp.float32),
    )(pooled_sum, W1, b1.reshape(1, HIDDEN), W2, b2.reshape(1, OUT))


def kernel(x, table, W1, b1, W2, b2):
    pooled_sum = _embed_pool(x, table)
    return _mlp(pooled_sum, W1, b1, W2, b2)


# speedup vs baseline: 13.4200x; 1.0971x over previous
import functools

import jax
import jax.numpy as jnp
from jax import lax
from jax.experimental import pallas as pl
from jax.experimental.pallas import tpu as pltpu
from jax.experimental.pallas import tpu_sc as plsc

VOCAB = 100000
EMBED = 128
HIDDEN = 2048
OUT = 1024
B = 16384
L = 50

NC = 2
NS = 16
NW = NC * NS
BPW = B // NW
NV = EMBED // 16
UNR = 10
NBUF = 8


BLK = 32
NBLK2 = BPW // (2 * BLK)


def _embed_pool_body(x_hbm, table_hbm, out_hbm, idx_v, rows_v, acc_v,
                     sem_g, sem_o):
    wid = lax.axis_index("s") * NC + lax.axis_index("c")
    base = wid * BPW

    pltpu.sync_copy(x_hbm.at[pl.ds(base, BPW)], idx_v)

    for j in range(NBUF):
        pltpu.make_async_copy(
            table_hbm.at[idx_v.at[j]], rows_v.at[j], sem_g).start()

    def blkbody(ib, carry):
        for k in range(2):
            blkbase = ib * (2 * BLK) + k * BLK

            @pl.when(ib > 0)
            def _(k=k):
                pltpu.make_async_copy(
                    acc_v.at[k], out_hbm.at[pl.ds(base, BLK)], sem_o).wait()

            def ibody(i2, c, k=k, blkbase=blkbase):
                for j in range(NBUF):
                    off = i2 * NBUF + j
                    bb = blkbase + off
                    pltpu.make_async_copy(
                        table_hbm.at[idx_v.at[0]], rows_v.at[j],
                        sem_g).wait()

                    def lbody(e, acc, j=j):
                        acc = list(acc)
                        for u in range(UNR):
                            for v in range(NV):
                                acc[v] = acc[v] + rows_v[j, e * UNR + u,
                                                         pl.ds(16 * v, 16)]
                        return acc

                    acc = lax.fori_loop(
                        0, L // UNR, lbody,
                        [jnp.zeros((16,), jnp.float32)] * NV)
                    for v in range(NV):
                        acc_v[k, off, pl.ds(16 * v, 16)] = acc[v]

                    @pl.when(bb + NBUF < BPW)
                    def _(j=j, bb=bb):
                        pltpu.make_async_copy(
                            table_hbm.at[idx_v.at[bb + NBUF]], rows_v.at[j],
                            sem_g).start()
                return c

            lax.fori_loop(0, BLK // NBUF, ibody, 0)

            pltpu.make_async_copy(
                acc_v.at[k], out_hbm.at[pl.ds(base + blkbase, BLK)],
                sem_o).start()
        return carry

    lax.fori_loop(0, NBLK2, blkbody, 0)

    for k in range(2):
        pltpu.make_async_copy(
            acc_v.at[k], out_hbm.at[pl.ds(base, BLK)], sem_o).wait()


_embed_pool = functools.partial(
    pl.kernel,
    mesh=plsc.VectorSubcoreMesh(core_axis_name="c", subcore_axis_name="s"),
    out_type=jax.ShapeDtypeStruct((B, EMBED), jnp.float32),
    scratch_types=[
        pltpu.VMEM((BPW, L), jnp.int32),
        pltpu.VMEM((NBUF, L, EMBED), jnp.float32),
        pltpu.VMEM((2, BLK, EMBED), jnp.float32),
        pltpu.SemaphoreType.DMA,
        pltpu.SemaphoreType.DMA,
    ],
)(_embed_pool_body)


BM = 512


def _mlp_kernel(x_ref, w1_ref, b1_ref, w2_ref, b2_ref, o_ref):
    x = x_ref[...] * (1.0 / L)
    h = jnp.dot(x, w1_ref[...], preferred_element_type=jnp.float32)
    h = jnp.maximum(h + b1_ref[...], 0.0)
    o = jnp.dot(h, w2_ref[...], preferred_element_type=jnp.float32)
    o_ref[...] = o + b2_ref[...]


def _mlp(pooled_sum, W1, b1, W2, b2):
    return pl.pallas_call(
        _mlp_kernel,
        grid=(B // BM,),
        in_specs=[
            pl.BlockSpec((BM, EMBED), lambda i: (i, 0)),
            pl.BlockSpec((EMBED, HIDDEN), lambda i: (0, 0)),
            pl.BlockSpec((1, HIDDEN), lambda i: (0, 0)),
            pl.BlockSpec((HIDDEN, OUT), lambda i: (0, 0)),
            pl.BlockSpec((1, OUT), lambda i: (0, 0)),
        ],
        out_specs=pl.BlockSpec((BM, OUT), lambda i: (i, 0)),
        out_shape=jax.ShapeDtypeStruct((B, OUT), jnp.float32),
    )(pooled_sum, W1, b1.reshape(1, HIDDEN), W2, b2.reshape(1, OUT))


def kernel(x, table, W1, b1, W2, b2):
    pooled_sum = _embed_pool(x, table)
    return _mlp(pooled_sum, W1, b1, W2, b2)


# speedup vs baseline: 14.4453x; 1.0764x over previous
import functools

import jax
import jax.numpy as jnp
from jax import lax
from jax.experimental import pallas as pl
from jax.experimental.pallas import tpu as pltpu
from jax.experimental.pallas import tpu_sc as plsc

VOCAB = 100000
EMBED = 128
HIDDEN = 2048
OUT = 1024
B = 16384
L = 50

NC = 2
NS = 16
NW = NC * NS
BPW = B // NW
NV = EMBED // 16
UNR = 10
NBUF = 8


BLK = 32
NBLK2 = BPW // (2 * BLK)


NSLICE = 4
SB = B // NSLICE
SPW = SB // NW
SNBLK2 = SPW // (2 * BLK)


def _embed_pool_body(x_hbm, table_hbm, out_hbm, idx_v, rows_v, acc_v,
                     sem_g, sem_o):
    wid = lax.axis_index("s") * NC + lax.axis_index("c")
    base = wid * SPW

    pltpu.sync_copy(x_hbm.at[pl.ds(base, SPW)], idx_v)

    for j in range(NBUF):
        pltpu.make_async_copy(
            table_hbm.at[idx_v.at[j]], rows_v.at[j], sem_g).start()

    def blkbody(ib, carry):
        for k in range(2):
            blkbase = ib * (2 * BLK) + k * BLK

            @pl.when(ib > 0)
            def _(k=k):
                pltpu.make_async_copy(
                    acc_v.at[k], out_hbm.at[pl.ds(base, BLK)], sem_o).wait()

            def ibody(i2, c, k=k, blkbase=blkbase):
                for j in range(NBUF):
                    off = i2 * NBUF + j
                    bb = blkbase + off
                    pltpu.make_async_copy(
                        table_hbm.at[idx_v.at[0]], rows_v.at[j],
                        sem_g).wait()

                    def lbody(e, acc, j=j):
                        acc = list(acc)
                        for u in range(UNR):
                            for v in range(NV):
                                acc[v] = acc[v] + rows_v[j, e * UNR + u,
                                                         pl.ds(16 * v, 16)]
                        return acc

                    acc = lax.fori_loop(
                        0, L // UNR, lbody,
                        [jnp.zeros((16,), jnp.float32)] * NV)
                    for v in range(NV):
                        acc_v[k, off, pl.ds(16 * v, 16)] = acc[v]

                    @pl.when(bb + NBUF < SPW)
                    def _(j=j, bb=bb):
                        pltpu.make_async_copy(
                            table_hbm.at[idx_v.at[bb + NBUF]], rows_v.at[j],
                            sem_g).start()
                return c

            lax.fori_loop(0, BLK // NBUF, ibody, 0)

            pltpu.make_async_copy(
                acc_v.at[k], out_hbm.at[pl.ds(base + blkbase, BLK)],
                sem_o).start()
        return carry

    lax.fori_loop(0, SNBLK2, blkbody, 0)

    for k in range(2):
        pltpu.make_async_copy(
            acc_v.at[k], out_hbm.at[pl.ds(base, BLK)], sem_o).wait()


_embed_pool = functools.partial(
    pl.kernel,
    mesh=plsc.VectorSubcoreMesh(core_axis_name="c", subcore_axis_name="s"),
    out_type=jax.ShapeDtypeStruct((SB, EMBED), jnp.float32),
    scratch_types=[
        pltpu.VMEM((SPW, L), jnp.int32),
        pltpu.VMEM((NBUF, L, EMBED), jnp.float32),
        pltpu.VMEM((2, BLK, EMBED), jnp.float32),
        pltpu.SemaphoreType.DMA,
        pltpu.SemaphoreType.DMA,
    ],
)(_embed_pool_body)


BM = 512
NSTEP = SB // BM


def _mlp_kernel(x_ref, w1_ref, b1_ref, w2_ref, b2_ref, o_ref):
    x = x_ref[...] * (1.0 / L)
    h = jnp.dot(x, w1_ref[...], preferred_element_type=jnp.float32)
    h = jnp.maximum(h + b1_ref[...], 0.0)
    o = jnp.dot(h, w2_ref[...], preferred_element_type=jnp.float32)
    o_ref[...] = o + b2_ref[...]


def _mlp_kernel_carry(carry_ref, x_ref, w1_ref, b1_ref, w2_ref, b2_ref,
                      o_ref):
    del carry_ref
    _mlp_kernel(x_ref, w1_ref, b1_ref, w2_ref, b2_ref, o_ref)


_WSPECS = [
    pl.BlockSpec((EMBED, HIDDEN), lambda i: (0, 0)),
    pl.BlockSpec((1, HIDDEN), lambda i: (0, 0)),
    pl.BlockSpec((HIDDEN, OUT), lambda i: (0, 0)),
    pl.BlockSpec((1, OUT), lambda i: (0, 0)),
]


def _mlp_slice(carry, pooled_sum, W1, b1, W2, b2, s):
    out_spec = pl.BlockSpec((BM, OUT), lambda i, s=s: (s * NSTEP + i, 0))
    out_shape = jax.ShapeDtypeStruct((B, OUT), jnp.float32)
    x_spec = pl.BlockSpec((BM, EMBED), lambda i: (i, 0))
    args = (pooled_sum, W1, b1.reshape(1, HIDDEN), W2, b2.reshape(1, OUT))
    if carry is None:
        return pl.pallas_call(
            _mlp_kernel,
            grid=(NSTEP,),
            in_specs=[x_spec] + _WSPECS,
            out_specs=out_spec,
            out_shape=out_shape,
        )(*args)
    return pl.pallas_call(
        _mlp_kernel_carry,
        grid=(NSTEP,),
        in_specs=[pl.BlockSpec(memory_space=pl.ANY), x_spec] + _WSPECS,
        out_specs=out_spec,
        out_shape=out_shape,
        input_output_aliases={0: 0},
    )(carry, *args)


def kernel(x, table, W1, b1, W2, b2):
    pooled = [
        _embed_pool(lax.slice_in_dim(x, s * SB, (s + 1) * SB), table)
        for s in range(NSLICE)
    ]
    out = None
    for s in range(NSLICE):
        out = _mlp_slice(out, pooled[s], W1, b1, W2, b2, s)
    return out
